# Initial kernel scaffold; baseline (speedup 1.0000x reference)
import jax
import jax.numpy as jnp
from jax.experimental import pallas as pl


def kernel(x, W, l1_centers, l2_centers, l3_centers):
    raise NotImplementedError("write your pallas kernel here")


# speedup vs baseline: 9.0220x; 9.0220x over previous
import functools

import jax
import jax.numpy as jnp
from jax.experimental import pallas as pl

N_EXPERTS = 64
N1, N2, N3 = 4, 4, 4
TOP_K = 8
K1 = 4
K2 = 8

_TILE = 512


def _argmin_topk(d, k, n):
    T = d.shape[0]
    iota = jax.lax.broadcasted_iota(jnp.int32, (T, n), 1)
    idxs = []
    dcur = d
    for _ in range(k):
        m = jnp.min(dcur, axis=1, keepdims=True)
        im = jnp.min(jnp.where(dcur == m, iota, n), axis=1, keepdims=True)
        idxs.append(im)
        dcur = jnp.where(iota == im, jnp.inf, dcur)
    return jnp.concatenate(idxs, axis=1)


def _routing_kernel(x_ref, wt_ref, ctr_ref, out_ref):
    xt = x_ref[...]
    wt = wt_ref[...]
    pos = jnp.dot(xt, wt, preferred_element_type=jnp.float32)
    T = xt.shape[0]
    px = pos[:, 0:1]
    py = pos[:, 1:2]
    pz = pos[:, 2:3]

    ctr = ctr_ref[...]
    l1x = ctr[0:1, 0:N1]
    l1y = ctr[1:2, 0:N1]
    l1z = ctr[2:3, 0:N1]
    d1 = jnp.sqrt(((px - l1x) ** 2 + (py - l1y) ** 2 + (pz - l1z) ** 2) + 1e-12)
    l1_idx = _argmin_topk(d1, K1, N1)

    l2x = ctr[0:1, 8:8 + N1 * N2]
    l2y = ctr[1:2, 8:8 + N1 * N2]
    l2z = ctr[2:3, 8:8 + N1 * N2]
    d2_all = (px - l2x) ** 2 + (py - l2y) ** 2 + (pz - l2z) ** 2

    l1k = jnp.concatenate(
        [jnp.broadcast_to(l1_idx[:, k:k + 1], (T, N2)) for k in range(K1)], axis=1)
    d2c = jnp.zeros((T, K1 * N2), jnp.float32)
    for c1 in range(N1):
        tile = jnp.concatenate([d2_all[:, c1 * N2:(c1 + 1) * N2]] * K1, axis=1)
        d2c = jnp.where(l1k == c1, tile, d2c)
    l2_flat = _argmin_topk(d2c, K2, K1 * N2)
    l1_parent = l2_flat // N2
    l2_local = l2_flat % N2
    l1_global = jnp.zeros((T, K2), jnp.int32)
    for p in range(K1):
        l1_global = jnp.where(l1_parent == p,
                              jnp.broadcast_to(l1_idx[:, p:p + 1], (T, K2)),
                              l1_global)
    l2_global = l1_global * N2 + l2_local

    l3x = ctr[0:1, 32:32 + 64]
    l3y = ctr[1:2, 32:32 + 64]
    l3z = ctr[2:3, 32:32 + 64]
    d3_all = (px - l3x) ** 2 + (py - l3y) ** 2 + (pz - l3z) ** 2

    l2k = jnp.concatenate(
        [jnp.broadcast_to(l2_global[:, k:k + 1], (T, N3)) for k in range(K2)], axis=1)
    d3c = jnp.zeros((T, K2 * N3), jnp.float32)
    for c in range(N1 * N2):
        tile = jnp.concatenate([d3_all[:, c * N3:(c + 1) * N3]] * K2, axis=1)
        d3c = jnp.where(l2k == c, tile, d3c)
    topk_flat = _argmin_topk(d3c, TOP_K, K2 * N3)
    l2_parent = topk_flat // N3
    l3_local = topk_flat % N3
    l2_parent_global = jnp.zeros((T, TOP_K), jnp.int32)
    for p in range(K2):
        l2_parent_global = jnp.where(l2_parent == p,
                                     jnp.broadcast_to(l2_global[:, p:p + 1], (T, TOP_K)),
                                     l2_parent_global)
    out_ref[...] = (l2_parent_global * N3 + l3_local) % N_EXPERTS


@functools.partial(jax.jit, static_argnames=())
def kernel(x, W, l1_centers, l2_centers, l3_centers):
    B, D = x.shape
    wt = jnp.zeros((D, 8), jnp.float32).at[:, :3].set(W.T)
    ctr = jnp.zeros((8, 128), jnp.float32)
    ctr = ctr.at[0:3, 0:N1].set(l1_centers.T)
    l2f = l2_centers.reshape(N1 * N2, 3)
    ctr = ctr.at[0:3, 8:8 + N1 * N2].set(l2f.T)
    l3f = l3_centers.reshape(N1 * N2 * N3, 3)
    ctr = ctr.at[0:3, 32:32 + 64].set(l3f.T)

    grid = (B // _TILE,)
    out = pl.pallas_call(
        _routing_kernel,
        grid=grid,
        in_specs=[
            pl.BlockSpec((_TILE, D), lambda i: (i, 0)),
            pl.BlockSpec((D, 8), lambda i: (0, 0)),
            pl.BlockSpec((8, 128), lambda i: (0, 0)),
        ],
        out_specs=pl.BlockSpec((_TILE, TOP_K), lambda i: (i, 0)),
        out_shape=jax.ShapeDtypeStruct((B, TOP_K), jnp.int32),
    )(x, wt, ctr)
    return out


# speedup vs baseline: 34.8130x; 3.8587x over previous
import functools

import jax
import jax.numpy as jnp
from jax.experimental import pallas as pl

N_EXPERTS = 64
N1, N2, N3 = 4, 4, 4
TOP_K = 8
K2 = 8

_TILE = 512


def _routing_kernel(x_ref, wt_ref, ctrT_ref, out_ref):
    xt = x_ref[...]
    wt = wt_ref[...]
    pos8 = jnp.dot(xt, wt, preferred_element_type=jnp.float32)
    T = xt.shape[0]
    posT = pos8.T
    px = posT[0:1, :]
    py = posT[1:2, :]
    pz = posT[2:3, :]

    ctrT = ctrT_ref[...]
    l3x = ctrT[:, 0:1]
    l3y = ctrT[:, 1:2]
    l3z = ctrT[:, 2:3]
    l2x = ctrT[0:16, 3:4]
    l2y = ctrT[0:16, 4:5]
    l2z = ctrT[0:16, 5:6]
    l1x = ctrT[0:4, 6:7]
    l1y = ctrT[0:4, 7:8]
    l1z = ctrT[0:4, 8:9]

    d1 = jnp.sqrt(((px - l1x) ** 2 + (py - l1y) ** 2 + (pz - l1z) ** 2) + 1e-12)
    rows = []
    for p in range(N1):
        dp = d1[p:p + 1, :]
        cnt = jnp.zeros_like(dp, dtype=jnp.int32)
        for q in range(N1):
            if q == p:
                continue
            lt = d1[q:q + 1, :] < dp
            if q < p:
                lt = lt | (d1[q:q + 1, :] == dp)
            cnt = cnt + lt.astype(jnp.int32)
        rows.append(cnt)
    rank1 = jnp.concatenate(rows, axis=0)

    d2 = (px - l2x) ** 2 + (py - l2y) ** 2 + (pz - l2z) ** 2
    iota16 = jax.lax.broadcasted_iota(jnp.int32, (16, T), 0)
    key2 = jnp.concatenate([rank1[p:p + 1, :] for p in range(N1) for _ in range(N2)],
                           axis=0) * N2 + (iota16 & 3)
    inf = jnp.float32(jnp.inf)
    l2g = []
    d2c = d2
    for _ in range(K2):
        m = jnp.min(d2c, axis=0, keepdims=True)
        eq = d2c == m
        km = jnp.min(jnp.where(eq, key2, 99), axis=0, keepdims=True)
        sel = eq & (key2 == km)
        c_star = jnp.min(jnp.where(sel, iota16, 64), axis=0, keepdims=True)
        l2g.append(c_star)
        d2c = jnp.where(iota16 == c_star, inf, d2c)

    iota64 = jax.lax.broadcasted_iota(jnp.int32, (64, T), 0)
    g_hi = iota64 >> 2
    g_lo = iota64 & 3
    key3 = jnp.full((64, T), 127, jnp.int32)
    for k in range(K2):
        key3 = jnp.where(g_hi == l2g[k], k * N3 + g_lo, key3)
    d3 = (px - l3x) ** 2 + (py - l3y) ** 2 + (pz - l3z) ** 2
    d3m = jnp.where(key3 < 64, d3, inf)
    outs = []
    for _ in range(TOP_K):
        m = jnp.min(d3m, axis=0, keepdims=True)
        eq = d3m == m
        km = jnp.min(jnp.where(eq, key3, 127), axis=0, keepdims=True)
        sel = eq & (key3 == km)
        g_star = jnp.min(jnp.where(sel, iota64, 64), axis=0, keepdims=True)
        outs.append(g_star)
        d3m = jnp.where(iota64 == g_star, inf, d3m)

    out = jnp.concatenate(outs, axis=0)
    out_ref[...] = out.T


@functools.partial(jax.jit, static_argnames=())
def kernel(x, W, l1_centers, l2_centers, l3_centers):
    B, D = x.shape
    wt = jnp.zeros((D, 8), jnp.float32).at[:, :3].set(W.T)
    ctrT = jnp.zeros((64, 128), jnp.float32)
    ctrT = ctrT.at[:, 0:3].set(l3_centers.reshape(64, 3))
    ctrT = ctrT.at[0:16, 3:6].set(l2_centers.reshape(16, 3))
    ctrT = ctrT.at[0:4, 6:9].set(l1_centers)

    grid = (B // _TILE,)
    out = pl.pallas_call(
        _routing_kernel,
        grid=grid,
        in_specs=[
            pl.BlockSpec((_TILE, D), lambda i: (i, 0)),
            pl.BlockSpec((D, 8), lambda i: (0, 0)),
            pl.BlockSpec((64, 128), lambda i: (0, 0)),
        ],
        out_specs=pl.BlockSpec((_TILE, TOP_K), lambda i: (i, 0)),
        out_shape=jax.ShapeDtypeStruct((B, TOP_K), jnp.int32),
    )(x, wt, ctrT)
    return out


# speedup vs baseline: 37.7205x; 1.0835x over previous
import functools

import jax
import jax.numpy as jnp
from jax.experimental import pallas as pl

N_EXPERTS = 64
N1, N2, N3 = 4, 4, 4
TOP_K = 8
K2 = 8

_TILE = 1024


def _routing_kernel(x_ref, wt_ref, ctrT_ref, out_ref):
    xt = x_ref[...]
    wt = wt_ref[...]
    pos8 = jnp.dot(xt, wt, preferred_element_type=jnp.float32)
    T = xt.shape[0]
    posT = pos8.T
    px = posT[0:1, :]
    py = posT[1:2, :]
    pz = posT[2:3, :]

    ctrT = ctrT_ref[...]
    l3x = ctrT[:, 0:1]
    l3y = ctrT[:, 1:2]
    l3z = ctrT[:, 2:3]
    l2x = ctrT[0:16, 3:4]
    l2y = ctrT[0:16, 4:5]
    l2z = ctrT[0:16, 5:6]
    l1x = ctrT[0:4, 6:7]
    l1y = ctrT[0:4, 7:8]
    l1z = ctrT[0:4, 8:9]

    d1 = jnp.sqrt(((px - l1x) ** 2 + (py - l1y) ** 2 + (pz - l1z) ** 2) + 1e-12)
    rows = []
    for p in range(N1):
        dp = d1[p:p + 1, :]
        cnt = jnp.zeros_like(dp, dtype=jnp.int32)
        for q in range(N1):
            if q == p:
                continue
            lt = d1[q:q + 1, :] < dp
            if q < p:
                lt = lt | (d1[q:q + 1, :] == dp)
            cnt = cnt + lt.astype(jnp.int32)
        rows.append(cnt)
    rank1 = jnp.concatenate(rows, axis=0)

    d2 = (px - l2x) ** 2 + (py - l2y) ** 2 + (pz - l2z) ** 2
    iota16 = jax.lax.broadcasted_iota(jnp.int32, (16, T), 0)
    key2 = jnp.concatenate([rank1[p:p + 1, :] for p in range(N1) for _ in range(N2)],
                           axis=0) * N2 + (iota16 & 3)
    inf = jnp.float32(jnp.inf)
    l2g = []
    d2c = d2
    for _ in range(K2):
        m = jnp.min(d2c, axis=0, keepdims=True)
        eq = d2c == m
        km = jnp.min(jnp.where(eq, key2, 99), axis=0, keepdims=True)
        sel = eq & (key2 == km)
        c_star = jnp.min(jnp.where(sel, iota16, 64), axis=0, keepdims=True)
        l2g.append(c_star)
        d2c = jnp.where(iota16 == c_star, inf, d2c)

    iota64 = jax.lax.broadcasted_iota(jnp.int32, (64, T), 0)
    g_hi = iota64 >> 2
    g_lo = iota64 & 3
    key3 = jnp.full((64, T), 127, jnp.int32)
    for k in range(K2):
        key3 = jnp.where(g_hi == l2g[k], k * N3 + g_lo, key3)
    d3 = (px - l3x) ** 2 + (py - l3y) ** 2 + (pz - l3z) ** 2
    d3m = jnp.where(key3 < 64, d3, inf)
    outs = []
    for _ in range(TOP_K):
        m = jnp.min(d3m, axis=0, keepdims=True)
        eq = d3m == m
        km = jnp.min(jnp.where(eq, key3, 127), axis=0, keepdims=True)
        sel = eq & (key3 == km)
        g_star = jnp.min(jnp.where(sel, iota64, 64), axis=0, keepdims=True)
        outs.append(g_star)
        d3m = jnp.where(iota64 == g_star, inf, d3m)

    out = jnp.concatenate(outs, axis=0)
    out_ref[...] = out.T


@functools.partial(jax.jit, static_argnames=())
def kernel(x, W, l1_centers, l2_centers, l3_centers):
    B, D = x.shape
    wt = jnp.zeros((D, 8), jnp.float32).at[:, :3].set(W.T)
    ctrT = jnp.zeros((64, 128), jnp.float32)
    ctrT = ctrT.at[:, 0:3].set(l3_centers.reshape(64, 3))
    ctrT = ctrT.at[0:16, 3:6].set(l2_centers.reshape(16, 3))
    ctrT = ctrT.at[0:4, 6:9].set(l1_centers)

    grid = (B // _TILE,)
    out = pl.pallas_call(
        _routing_kernel,
        grid=grid,
        in_specs=[
            pl.BlockSpec((_TILE, D), lambda i: (i, 0)),
            pl.BlockSpec((D, 8), lambda i: (0, 0)),
            pl.BlockSpec((64, 128), lambda i: (0, 0)),
        ],
        out_specs=pl.BlockSpec((_TILE, TOP_K), lambda i: (i, 0)),
        out_shape=jax.ShapeDtypeStruct((B, TOP_K), jnp.int32),
    )(x, wt, ctrT)
    return out


# speedup vs baseline: 37.8102x; 1.0024x over previous
import functools

import jax
import jax.numpy as jnp
from jax.experimental import pallas as pl

N_EXPERTS = 64
N1, N2, N3 = 4, 4, 4
TOP_K = 8
K2 = 8

_TILE = 2048


def _routing_kernel(x_ref, wt_ref, ctrT_ref, out_ref):
    xt = x_ref[...]
    wt = wt_ref[...]
    pos8 = jnp.dot(xt, wt, preferred_element_type=jnp.float32)
    T = xt.shape[0]
    posT = pos8.T
    px = posT[0:1, :]
    py = posT[1:2, :]
    pz = posT[2:3, :]

    ctrT = ctrT_ref[...]
    l3x = ctrT[:, 0:1]
    l3y = ctrT[:, 1:2]
    l3z = ctrT[:, 2:3]
    l2x = ctrT[0:16, 3:4]
    l2y = ctrT[0:16, 4:5]
    l2z = ctrT[0:16, 5:6]
    l1x = ctrT[0:4, 6:7]
    l1y = ctrT[0:4, 7:8]
    l1z = ctrT[0:4, 8:9]

    d1 = jnp.sqrt(((px - l1x) ** 2 + (py - l1y) ** 2 + (pz - l1z) ** 2) + 1e-12)
    rows = []
    for p in range(N1):
        dp = d1[p:p + 1, :]
        cnt = jnp.zeros_like(dp, dtype=jnp.int32)
        for q in range(N1):
            if q == p:
                continue
            lt = d1[q:q + 1, :] < dp
            if q < p:
                lt = lt | (d1[q:q + 1, :] == dp)
            cnt = cnt + lt.astype(jnp.int32)
        rows.append(cnt)
    rank1 = jnp.concatenate(rows, axis=0)

    d2 = (px - l2x) ** 2 + (py - l2y) ** 2 + (pz - l2z) ** 2
    iota16 = jax.lax.broadcasted_iota(jnp.int32, (16, T), 0)
    key2 = jnp.concatenate([rank1[p:p + 1, :] for p in range(N1) for _ in range(N2)],
                           axis=0) * N2 + (iota16 & 3)
    inf = jnp.float32(jnp.inf)
    l2g = []
    d2c = d2
    for _ in range(K2):
        m = jnp.min(d2c, axis=0, keepdims=True)
        eq = d2c == m
        km = jnp.min(jnp.where(eq, key2, 99), axis=0, keepdims=True)
        sel = eq & (key2 == km)
        c_star = jnp.min(jnp.where(sel, iota16, 64), axis=0, keepdims=True)
        l2g.append(c_star)
        d2c = jnp.where(iota16 == c_star, inf, d2c)

    iota64 = jax.lax.broadcasted_iota(jnp.int32, (64, T), 0)
    g_hi = iota64 >> 2
    g_lo = iota64 & 3
    key3 = jnp.full((64, T), 127, jnp.int32)
    for k in range(K2):
        key3 = jnp.where(g_hi == l2g[k], k * N3 + g_lo, key3)
    d3 = (px - l3x) ** 2 + (py - l3y) ** 2 + (pz - l3z) ** 2
    d3m = jnp.where(key3 < 64, d3, inf)
    outs = []
    for _ in range(TOP_K):
        m = jnp.min(d3m, axis=0, keepdims=True)
        eq = d3m == m
        km = jnp.min(jnp.where(eq, key3, 127), axis=0, keepdims=True)
        sel = eq & (key3 == km)
        g_star = jnp.min(jnp.where(sel, iota64, 64), axis=0, keepdims=True)
        outs.append(g_star)
        d3m = jnp.where(iota64 == g_star, inf, d3m)

    out = jnp.concatenate(outs, axis=0)
    out_ref[...] = out.T


@functools.partial(jax.jit, static_argnames=())
def kernel(x, W, l1_centers, l2_centers, l3_centers):
    B, D = x.shape
    wt = jnp.zeros((D, 8), jnp.float32).at[:, :3].set(W.T)
    ctrT = jnp.zeros((64, 128), jnp.float32)
    ctrT = ctrT.at[:, 0:3].set(l3_centers.reshape(64, 3))
    ctrT = ctrT.at[0:16, 3:6].set(l2_centers.reshape(16, 3))
    ctrT = ctrT.at[0:4, 6:9].set(l1_centers)

    grid = (B // _TILE,)
    out = pl.pallas_call(
        _routing_kernel,
        grid=grid,
        in_specs=[
            pl.BlockSpec((_TILE, D), lambda i: (i, 0)),
            pl.BlockSpec((D, 8), lambda i: (0, 0)),
            pl.BlockSpec((64, 128), lambda i: (0, 0)),
        ],
        out_specs=pl.BlockSpec((_TILE, TOP_K), lambda i: (i, 0)),
        out_shape=jax.ShapeDtypeStruct((B, TOP_K), jnp.int32),
    )(x, wt, ctrT)
    return out
